# baseline (speedup 1.0000x reference)
import functools

import jax
import jax.numpy as jnp
from jax import lax
from jax.experimental import pallas as pl
from jax.experimental.pallas import tpu as pltpu
from jax.experimental.pallas import tpu_sc as plsc

_RADIUS = 0.1
_K = 16
_STRIDE = 4


def _topk_body(pos_ref, cposT_ref, gidx_ref, sidx_ref, nbrd_ref):
    b = pl.program_id(0)
    p = pos_ref[0]
    cT = cposT_ref[0]
    n = p.shape[0]
    x, y, z = p[:, 0:1], p[:, 1:2], p[:, 2:3]
    cx, cy, cz = cT[0:1, :], cT[1:2, :], cT[2:3, :]
    sq = (x * x + y * y) + z * z
    csq = (cx * cx + cy * cy) + cz * cz
    dot = jnp.dot(p, cT, preferred_element_type=jnp.float32)
    d2 = (sq + csq) - 2.0 * dot
    d2 = jnp.maximum(d2, 0.0)
    kx = jnp.floor(x / _RADIUS)
    ky = jnp.floor(y / _RADIUS)
    kz = jnp.floor(z / _RADIUS)
    ckx = jnp.floor(cx / _RADIUS)
    cky = jnp.floor(cy / _RADIUS)
    ckz = jnp.floor(cz / _RADIUS)
    cheb = jnp.maximum(jnp.maximum(jnp.abs(kx - ckx), jnp.abs(ky - cky)),
                       jnp.abs(kz - ckz))
    vals = jnp.where(cheb <= 1.0, d2, 1e9)
    iota0 = lax.broadcasted_iota(jnp.int32, vals.shape, 0).astype(jnp.float32)
    for k in range(_K):
        m = jnp.min(vals, axis=0, keepdims=True)
        cand = jnp.where(vals == m, iota0, jnp.float32(n))
        ji = jnp.min(cand, axis=0, keepdims=True)
        jii = ji.astype(jnp.int32)
        nbrd_ref[0, k:k + 1, :] = m
        sidx_ref[0, k:k + 1, :] = jii
        gidx_ref[0, k:k + 1, :] = jii + b * n
        vals = jnp.where(iota0 == ji, jnp.float32(jnp.inf), vals)


def _masked_topk(pos, cposT, n_sample):
    B, N, _ = pos.shape
    spec3 = pl.BlockSpec((1, _K, n_sample), lambda b: (b, 0, 0))
    return pl.pallas_call(
        _topk_body,
        grid=(B,),
        in_specs=[pl.BlockSpec((1, N, 3), lambda b: (b, 0, 0)),
                  pl.BlockSpec((1, 3, n_sample), lambda b: (b, 0, 0))],
        out_specs=[spec3, spec3, spec3],
        out_shape=[jax.ShapeDtypeStruct((B, _K, n_sample), jnp.int32),
                   jax.ShapeDtypeStruct((B, _K, n_sample), jnp.int32),
                   jax.ShapeDtypeStruct((B, _K, n_sample), jnp.float32)],
    )(pos, cposT)


def _sc_gather_concat(table, idx, dvals):
    R, C = table.shape
    E = idx.shape[0]
    W = C + 1
    info = plsc.get_sparse_core_info()
    nc, ns = info.num_cores, info.num_subcores
    nw = nc * ns
    per_w = E // nw
    chunk = 128
    n_chunks = per_w // chunk
    mesh = plsc.VectorSubcoreMesh(core_axis_name="c", subcore_axis_name="s")

    @functools.partial(
        pl.kernel,
        mesh=mesh,
        out_type=jax.ShapeDtypeStruct((E * W,), jnp.float32),
        scratch_types=[
            pltpu.VMEM((chunk,), jnp.int32),
            pltpu.VMEM((chunk + 16,), jnp.float32),
            pltpu.VMEM((chunk, C), jnp.float32),
            pltpu.VMEM((chunk * W + 16,), jnp.float32),
            pltpu.SemaphoreType.DMA,
        ],
    )
    def gather_kernel(table_hbm, idx_hbm, d_hbm, out_hbm, idx_v, d_v,
                      rows_v, rows_w, sem):
        wid = lax.axis_index("s") * nc + lax.axis_index("c")
        base = wid * per_w

        def body(c, carry):
            off = base + c * chunk
            pltpu.sync_copy(idx_hbm.at[pl.ds(off, chunk)], idx_v)
            pltpu.sync_copy(d_hbm.at[pl.ds(off, chunk)],
                            d_v.at[pl.ds(0, chunk)])
            pltpu.async_copy(table_hbm.at[idx_v], rows_v, sem).wait()

            def rowbody(r, carry2):
                rb = r * W
                for g in range(C // 16):
                    rows_w[pl.ds(rb + g * 16, 16)] = rows_v[r, pl.ds(g * 16, 16)]
                rows_w[pl.ds(rb + C, 16)] = d_v[pl.ds(r, 16)]
                return carry2

            lax.fori_loop(0, chunk, rowbody, 0)
            pltpu.sync_copy(rows_w.at[pl.ds(0, chunk * W)],
                            out_hbm.at[pl.ds(off * W, chunk * W)])
            return carry

        lax.fori_loop(0, n_chunks, body, 0)

    return gather_kernel(table, idx, dvals)


def kernel(pos, feat):
    B, N, D = pos.shape
    C = feat.shape[-1]
    n_sample = N // _STRIDE
    cposT = jnp.transpose(pos[:, ::_STRIDE, :], (0, 2, 1))
    gidx, sidx, nbrd = _masked_topk(pos, cposT, n_sample)
    src_idx = jnp.transpose(sidx, (0, 2, 1))
    flat_idx = jnp.transpose(gidx, (0, 2, 1)).reshape(-1)
    flat_d = jnp.transpose(nbrd, (0, 2, 1)).reshape(-1)
    out = _sc_gather_concat(feat.reshape(B * N, C), flat_idx, flat_d)
    out = out.reshape(B, n_sample, _K, C + 1)
    centers = (jnp.arange(n_sample, dtype=jnp.int32) * _STRIDE)
    dst_idx = jnp.broadcast_to(centers[None, :, None], src_idx.shape)
    return out, src_idx, dst_idx


# speedup vs baseline: 1.5190x; 1.5190x over previous
import functools

import jax
import jax.numpy as jnp
from jax import lax
from jax.experimental import pallas as pl
from jax.experimental.pallas import tpu as pltpu
from jax.experimental.pallas import tpu_sc as plsc

_RADIUS = 0.1
_K = 16
_STRIDE = 4


def _topk_body(pos_ref, gidx_ref, sidx_ref, nbrd_ref):
    b = pl.program_id(0)
    p = pos_ref[0]
    n = p.shape[0]
    c3 = p.reshape(n // _STRIDE, _STRIDE, 3)[:, 0, :]
    cT = jnp.transpose(c3, (1, 0))
    x, y, z = p[:, 0:1], p[:, 1:2], p[:, 2:3]
    cx, cy, cz = cT[0:1, :], cT[1:2, :], cT[2:3, :]
    sq = (x * x + y * y) + z * z
    csq = (cx * cx + cy * cy) + cz * cz
    dot = jnp.dot(p, cT, preferred_element_type=jnp.float32)
    d2 = (sq + csq) - 2.0 * dot
    d2 = jnp.maximum(d2, 0.0)
    kx = jnp.floor(x / _RADIUS)
    ky = jnp.floor(y / _RADIUS)
    kz = jnp.floor(z / _RADIUS)
    ckx = jnp.floor(cx / _RADIUS)
    cky = jnp.floor(cy / _RADIUS)
    ckz = jnp.floor(cz / _RADIUS)
    cheb = jnp.maximum(jnp.maximum(jnp.abs(kx - ckx), jnp.abs(ky - cky)),
                       jnp.abs(kz - ckz))
    vals = jnp.where(cheb <= 1.0, d2, 1e9)
    iota0 = lax.broadcasted_iota(jnp.int32, vals.shape, 0).astype(jnp.float32)
    for k in range(_K):
        m = jnp.min(vals, axis=0, keepdims=True)
        cand = jnp.where(vals == m, iota0, jnp.float32(n))
        ji = jnp.min(cand, axis=0, keepdims=True)
        jii = ji.astype(jnp.int32)
        nbrd_ref[0, k:k + 1, :] = m
        sidx_ref[0, k:k + 1, :] = jii
        gidx_ref[0, k:k + 1, :] = jii + b * n
        vals = jnp.where(iota0 == ji, jnp.float32(jnp.inf), vals)


def _masked_topk(pos, n_sample):
    B, N, _ = pos.shape
    spec3 = pl.BlockSpec((1, _K, n_sample), lambda b: (b, 0, 0))
    return pl.pallas_call(
        _topk_body,
        grid=(B,),
        in_specs=[pl.BlockSpec((1, N, 3), lambda b: (b, 0, 0))],
        out_specs=[spec3, spec3, spec3],
        out_shape=[jax.ShapeDtypeStruct((B, _K, n_sample), jnp.int32),
                   jax.ShapeDtypeStruct((B, _K, n_sample), jnp.int32),
                   jax.ShapeDtypeStruct((B, _K, n_sample), jnp.float32)],
    )(pos)


def _sc_gather(table, idx):
    R, C = table.shape
    E = idx.shape[0]
    info = plsc.get_sparse_core_info()
    nc, ns = info.num_cores, info.num_subcores
    nw = nc * ns
    per_w = E // nw
    chunk = 128
    n_chunks = per_w // chunk
    mesh = plsc.VectorSubcoreMesh(core_axis_name="c", subcore_axis_name="s")

    @functools.partial(
        pl.kernel,
        mesh=mesh,
        out_type=jax.ShapeDtypeStruct((E, C), jnp.float32),
        scratch_types=[
            pltpu.VMEM((chunk,), jnp.int32),
            pltpu.VMEM((chunk, C), jnp.float32),
            pltpu.SemaphoreType.DMA,
        ],
    )
    def gather_kernel(table_hbm, idx_hbm, out_hbm, idx_v, rows_v, sem):
        wid = lax.axis_index("s") * nc + lax.axis_index("c")
        base = wid * per_w

        def body(c, carry):
            off = base + c * chunk
            pltpu.sync_copy(idx_hbm.at[pl.ds(off, chunk)], idx_v)
            pltpu.async_copy(table_hbm.at[idx_v], rows_v, sem).wait()
            pltpu.sync_copy(rows_v, out_hbm.at[pl.ds(off, chunk)])
            return carry

        lax.fori_loop(0, n_chunks, body, 0)

    return gather_kernel(table, idx)


def kernel(pos, feat):
    B, N, D = pos.shape
    C = feat.shape[-1]
    n_sample = N // _STRIDE
    gidx, sidx, nbrd = _masked_topk(pos, n_sample)
    src_idx = jnp.transpose(sidx, (0, 2, 1))
    nbr_d = jnp.transpose(nbrd, (0, 2, 1))
    flat_idx = jnp.transpose(gidx, (0, 2, 1)).reshape(-1)
    nbr_feat = _sc_gather(feat.reshape(B * N, C), flat_idx)
    nbr_feat = nbr_feat.reshape(B, n_sample, _K, C)
    out = jnp.concatenate([nbr_feat, nbr_d[..., None]], axis=-1)
    centers = (jnp.arange(n_sample, dtype=jnp.int32) * _STRIDE)
    dst_idx = jnp.broadcast_to(centers[None, :, None], src_idx.shape)
    return out, src_idx, dst_idx


# speedup vs baseline: 1.5986x; 1.0524x over previous
import functools

import jax
import jax.numpy as jnp
from jax import lax
from jax.experimental import pallas as pl
from jax.experimental.pallas import tpu as pltpu
from jax.experimental.pallas import tpu_sc as plsc

_RADIUS = 0.1
_K = 16
_STRIDE = 4


def _topk_body(pos_ref, gidx_ref, sidx_ref, nbrd_ref):
    b = pl.program_id(0)
    p = pos_ref[0]
    n = p.shape[0]
    c3 = p.reshape(n // _STRIDE, _STRIDE, 3)[:, 0, :]
    cT = jnp.transpose(c3, (1, 0))
    x, y, z = p[:, 0:1], p[:, 1:2], p[:, 2:3]
    cx, cy, cz = cT[0:1, :], cT[1:2, :], cT[2:3, :]
    sq = (x * x + y * y) + z * z
    csq = (cx * cx + cy * cy) + cz * cz
    dot = jnp.dot(p, cT, preferred_element_type=jnp.float32)
    d2 = (sq + csq) - 2.0 * dot
    d2 = jnp.maximum(d2, 0.0)
    kx = jnp.floor(x / _RADIUS)
    ky = jnp.floor(y / _RADIUS)
    kz = jnp.floor(z / _RADIUS)
    ckx = jnp.floor(cx / _RADIUS)
    cky = jnp.floor(cy / _RADIUS)
    ckz = jnp.floor(cz / _RADIUS)
    cheb = jnp.maximum(jnp.maximum(jnp.abs(kx - ckx), jnp.abs(ky - cky)),
                       jnp.abs(kz - ckz))
    vals = jnp.where(cheb <= 1.0, d2, 1e9)
    iota0 = lax.broadcasted_iota(jnp.int32, vals.shape, 0).astype(jnp.float32)
    for k in range(_K):
        m = jnp.min(vals, axis=0, keepdims=True)
        cand = jnp.where(vals == m, iota0, jnp.float32(n))
        ji = jnp.min(cand, axis=0, keepdims=True)
        jii = ji.astype(jnp.int32)
        nbrd_ref[0, k:k + 1, :] = m
        sidx_ref[0, k:k + 1, :] = jii
        gidx_ref[0, k:k + 1, :] = jii + b * n
        vals = jnp.where(iota0 == ji, jnp.float32(jnp.inf), vals)


def _masked_topk(pos, n_sample):
    B, N, _ = pos.shape
    spec3 = pl.BlockSpec((1, _K, n_sample), lambda b: (b, 0, 0))
    return pl.pallas_call(
        _topk_body,
        grid=(B,),
        in_specs=[pl.BlockSpec((1, N, 3), lambda b: (b, 0, 0))],
        out_specs=[spec3, spec3, spec3],
        out_shape=[jax.ShapeDtypeStruct((B, _K, n_sample), jnp.int32),
                   jax.ShapeDtypeStruct((B, _K, n_sample), jnp.int32),
                   jax.ShapeDtypeStruct((B, _K, n_sample), jnp.float32)],
    )(pos)


def _sc_gather(table, idx):
    R, C = table.shape
    E = idx.shape[0]
    info = plsc.get_sparse_core_info()
    nc, ns = info.num_cores, info.num_subcores
    nw = nc * ns
    per_w = E // nw
    chunk = 128
    n_chunks = per_w // chunk
    mesh = plsc.VectorSubcoreMesh(core_axis_name="c", subcore_axis_name="s")

    @functools.partial(
        pl.kernel,
        mesh=mesh,
        out_type=jax.ShapeDtypeStruct((E, C), jnp.float32),
        scratch_types=[
            pltpu.VMEM((per_w,), jnp.int32),
            pltpu.VMEM((chunk, C), jnp.float32),
            pltpu.VMEM((chunk, C), jnp.float32),
            pltpu.SemaphoreType.DMA,
            pltpu.SemaphoreType.DMA,
        ],
    )
    def gather_kernel(table_hbm, idx_hbm, out_hbm, idx_v, rows_a, rows_b,
                      sem_a, sem_b):
        wid = lax.axis_index("s") * nc + lax.axis_index("c")
        base = wid * per_w
        pltpu.sync_copy(idx_hbm.at[pl.ds(base, per_w)], idx_v)
        bufs = (rows_a, rows_b)
        sems = (sem_a, sem_b)

        def start(c):
            return pltpu.async_copy(
                table_hbm.at[idx_v.at[pl.ds(c * chunk, chunk)]],
                bufs[c % 2], sems[c % 2])

        cp = start(0)
        for c in range(n_chunks):
            nxt = start(c + 1) if c + 1 < n_chunks else None
            cp.wait()
            pltpu.sync_copy(bufs[c % 2],
                            out_hbm.at[pl.ds(base + c * chunk, chunk)])
            cp = nxt

    return gather_kernel(table, idx)


def kernel(pos, feat):
    B, N, D = pos.shape
    C = feat.shape[-1]
    n_sample = N // _STRIDE
    gidx, sidx, nbrd = _masked_topk(pos, n_sample)
    src_idx = jnp.transpose(sidx, (0, 2, 1))
    nbr_d = jnp.transpose(nbrd, (0, 2, 1))
    flat_idx = jnp.transpose(gidx, (0, 2, 1)).reshape(-1)
    nbr_feat = _sc_gather(feat.reshape(B * N, C), flat_idx)
    nbr_feat = nbr_feat.reshape(B, n_sample, _K, C)
    out = jnp.concatenate([nbr_feat, nbr_d[..., None]], axis=-1)
    centers = (jnp.arange(n_sample, dtype=jnp.int32) * _STRIDE)
    dst_idx = jnp.broadcast_to(centers[None, :, None], src_idx.shape)
    return out, src_idx, dst_idx
